# baseline (speedup 1.0000x reference)
import jax
import jax.numpy as jnp
from jax import lax
from jax.experimental import pallas as pl
from jax.experimental.pallas import tpu as pltpu
from jax.experimental.pallas import tpu_sc as plsc

V = 10000
WEIGHT = 0.1
NC, NS, L = 2, 16, 16
NW = NC * NS
EPW = 3840
E_PAD = NW * EPW
T_SIZE = V * V + 8
DUMMY = V
ACC_ROWS = 10112
ACC_F = 4 * ACC_ROWS
VPAD = 10016


def _fill_keys(rows_v, cols_v, keys_v):
    @pl.loop(0, EPW // L)
    def _(i):
        sl = pl.ds(i * L, L)
        keys_v[sl] = rows_v[sl] * V + cols_v[sl]


def _k1_body(rows_hbm, cols_hbm, t_hbm, rows_v, cols_v, keys_v, eid_v, sem):
    wid = lax.axis_index("s") * NC + lax.axis_index("c")
    pltpu.sync_copy(rows_hbm.at[wid], rows_v)
    pltpu.sync_copy(cols_hbm.at[wid], cols_v)
    _fill_keys(rows_v, cols_v, keys_v)
    base = wid * EPW
    iota = lax.iota(jnp.int32, L)

    @pl.loop(0, EPW // L)
    def _(i):
        eid_v[pl.ds(i * L, L)] = base + i * L + iota

    pltpu.async_copy(eid_v, t_hbm.at[keys_v], sem).wait()


def _k2_body(rows_hbm, cols_hbm, t_hbm, verts_hbm, zeros_hbm, out_hbm,
             rows_v, cols_v, keys_v, w_v, verts_v, acc_v, sem):
    wid = lax.axis_index("s") * NC + lax.axis_index("c")
    pltpu.sync_copy(rows_hbm.at[wid], rows_v)
    pltpu.sync_copy(cols_hbm.at[wid], cols_v)
    pltpu.sync_copy(verts_hbm, verts_v)
    pltpu.sync_copy(zeros_hbm, acc_v)

    _fill_keys(rows_v, cols_v, keys_v)
    pltpu.async_copy(t_hbm.at[keys_v], w_v, sem).wait()

    base = wid * EPW
    iota = lax.iota(jnp.int32, L)
    ones = jnp.ones((L,), jnp.float32)

    @pl.loop(0, EPW // L)
    def _(i):
        sl = pl.ds(i * L, L)
        r = rows_v[sl]
        c = cols_v[sl]
        w = w_v[sl]
        eid = base + i * L + iota
        canonical = w == eid
        rr = jnp.where(canonical, r, DUMMY)
        for k in range(3):
            val = plsc.load_gather(verts_v, [jnp.full((L,), k, jnp.int32), c])
            plsc.addupdate_scatter(acc_v, [rr + k * ACC_ROWS], val)
        plsc.addupdate_scatter(acc_v, [rr + 3 * ACC_ROWS], ones)

    pltpu.sync_copy(acc_v, out_hbm.at[wid])


def _k3_body(partials_ref, verts_ref, out_ref):
    a = jnp.sum(partials_ref[...], axis=0, keepdims=True)
    deg = a[:, 3 * ACC_ROWS:4 * ACC_ROWS]
    valid = lax.broadcasted_iota(jnp.int32, (1, ACC_ROWS), 1) < V
    total = jnp.zeros((), jnp.float32)
    for k in range(3):
        s = a[:, k * ACC_ROWS:(k + 1) * ACC_ROWS]
        vk = verts_ref[k:k + 1, :]
        r = jnp.where(valid, deg * vk - s, 0.0)
        total = total + jnp.sum(r * r)
    out_ref[...] = jnp.full((1, 1), (WEIGHT / V) * total, jnp.float32)


@jax.jit
def kernel(vertices, faces):
    src_sel = jnp.array([0, 0, 1, 1, 2, 2])
    dst_sel = jnp.array([1, 2, 0, 2, 0, 1])
    rows = faces[:, src_sel].reshape(-1).astype(jnp.int32)
    cols = faces[:, dst_sel].reshape(-1).astype(jnp.int32)
    e = rows.shape[0]
    rows2 = jnp.full((E_PAD,), V, jnp.int32).at[:e].set(rows).reshape(NW, EPW)
    cols2 = jnp.zeros((E_PAD,), jnp.int32).at[:e].set(cols).reshape(NW, EPW)
    verts = vertices[0].astype(jnp.float32)
    verts_t = jnp.zeros((3, VPAD), jnp.float32).at[:, :V].set(verts.T)
    verts_t3 = jnp.zeros((3, ACC_ROWS), jnp.float32).at[:, :V].set(verts.T)
    acc_zeros = jnp.zeros((ACC_F,), jnp.float32)

    mesh = plsc.VectorSubcoreMesh(core_axis_name="c", subcore_axis_name="s")
    sc_params = pltpu.CompilerParams(
        use_tc_tiling_on_sc=False, needs_layout_passes=False)

    k1 = pl.kernel(
        _k1_body,
        out_type=jax.ShapeDtypeStruct((T_SIZE,), jnp.int32),
        mesh=mesh,
        compiler_params=sc_params,
        scratch_types=[
            pltpu.VMEM((EPW,), jnp.int32),
            pltpu.VMEM((EPW,), jnp.int32),
            pltpu.VMEM((EPW,), jnp.int32),
            pltpu.VMEM((EPW,), jnp.int32),
            pltpu.SemaphoreType.DMA,
        ],
    )
    table = k1(rows2, cols2)

    k2 = pl.kernel(
        _k2_body,
        out_type=jax.ShapeDtypeStruct((NW, ACC_F), jnp.float32),
        mesh=mesh,
        compiler_params=sc_params,
        scratch_types=[
            pltpu.VMEM((EPW,), jnp.int32),
            pltpu.VMEM((EPW,), jnp.int32),
            pltpu.VMEM((EPW,), jnp.int32),
            pltpu.VMEM((EPW,), jnp.int32),
            pltpu.VMEM((3, VPAD), jnp.float32),
            pltpu.VMEM((ACC_F,), jnp.float32),
            pltpu.SemaphoreType.DMA,
        ],
    )
    partials = k2(rows2, cols2, table, verts_t, acc_zeros)

    out = pl.pallas_call(
        _k3_body,
        out_shape=jax.ShapeDtypeStruct((1, 1), jnp.float32),
    )(partials, verts_t3)
    return out[0, 0]


# speedup vs baseline: 1.7838x; 1.7838x over previous
import jax
import jax.numpy as jnp
from jax import lax
from jax.experimental import pallas as pl
from jax.experimental.pallas import tpu as pltpu
from jax.experimental.pallas import tpu_sc as plsc

V = 10000
WEIGHT = 0.1
NC, NS, L = 2, 16, 16
NW = NC * NS
EPW = 1920
E_PAD = NW * EPW
T_SIZE = V * V + V + 8
DUMMY = V
ACC_ROWS = 10112
ACC_F = 4 * ACC_ROWS
VPAD = 10016


def _fill_keys(rows_v, cols_v, keys_v):
    @pl.loop(0, EPW // L)
    def _(i):
        sl = pl.ds(i * L, L)
        r = rows_v[sl]
        c = cols_v[sl]
        keys_v[sl] = jnp.minimum(r, c) * V + jnp.maximum(r, c)


def _k1_body(rows_hbm, cols_hbm, t_hbm, rows_v, cols_v, keys_v, eid_v, sem):
    wid = lax.axis_index("s") * NC + lax.axis_index("c")
    pltpu.sync_copy(rows_hbm.at[wid], rows_v)
    pltpu.sync_copy(cols_hbm.at[wid], cols_v)
    _fill_keys(rows_v, cols_v, keys_v)
    base = wid * EPW
    iota = lax.iota(jnp.int32, L)

    @pl.loop(0, EPW // L)
    def _(i):
        eid_v[pl.ds(i * L, L)] = base + i * L + iota

    pltpu.async_copy(eid_v, t_hbm.at[keys_v], sem).wait()


def _k2_body(rows_hbm, cols_hbm, t_hbm, verts_hbm, zeros_hbm, out_hbm,
             rows_v, cols_v, keys_v, w_v, verts_v, acc_v, sem):
    wid = lax.axis_index("s") * NC + lax.axis_index("c")
    pltpu.sync_copy(rows_hbm.at[wid], rows_v)
    pltpu.sync_copy(cols_hbm.at[wid], cols_v)
    pltpu.sync_copy(verts_hbm, verts_v)
    pltpu.sync_copy(zeros_hbm, acc_v)

    _fill_keys(rows_v, cols_v, keys_v)
    pltpu.async_copy(t_hbm.at[keys_v], w_v, sem).wait()

    base = wid * EPW
    iota = lax.iota(jnp.int32, L)
    ones = jnp.ones((L,), jnp.float32)

    @pl.loop(0, EPW // L)
    def _(i):
        sl = pl.ds(i * L, L)
        r = rows_v[sl]
        c = cols_v[sl]
        w = w_v[sl]
        eid = base + i * L + iota
        canonical = w == eid
        rr = jnp.where(canonical, r, DUMMY)
        cc = jnp.where(canonical & (r != c), c, DUMMY)
        for k in range(3):
            kf = jnp.full((L,), k, jnp.int32)
            vc = plsc.load_gather(verts_v, [kf, c])
            plsc.addupdate_scatter(acc_v, [rr + k * ACC_ROWS], vc)
            vr = plsc.load_gather(verts_v, [kf, r])
            plsc.addupdate_scatter(acc_v, [cc + k * ACC_ROWS], vr)
        plsc.addupdate_scatter(acc_v, [rr + 3 * ACC_ROWS], ones)
        plsc.addupdate_scatter(acc_v, [cc + 3 * ACC_ROWS], ones)

    pltpu.sync_copy(acc_v, out_hbm.at[wid])


def _k3_body(partials_ref, verts_ref, out_ref):
    a = jnp.sum(partials_ref[...], axis=0, keepdims=True)
    deg = a[:, 3 * ACC_ROWS:4 * ACC_ROWS]
    valid = lax.broadcasted_iota(jnp.int32, (1, ACC_ROWS), 1) < V
    total = jnp.zeros((), jnp.float32)
    for k in range(3):
        s = a[:, k * ACC_ROWS:(k + 1) * ACC_ROWS]
        vk = verts_ref[k:k + 1, :]
        r = jnp.where(valid, deg * vk - s, 0.0)
        total = total + jnp.sum(r * r)
    out_ref[...] = jnp.full((1, 1), (WEIGHT / V) * total, jnp.float32)


@jax.jit
def kernel(vertices, faces):
    src_sel = jnp.array([0, 0, 1])
    dst_sel = jnp.array([1, 2, 2])
    rows = faces[:, src_sel].reshape(-1).astype(jnp.int32)
    cols = faces[:, dst_sel].reshape(-1).astype(jnp.int32)
    e = rows.shape[0]
    rows2 = jnp.full((E_PAD,), V, jnp.int32).at[:e].set(rows).reshape(NW, EPW)
    cols2 = jnp.full((E_PAD,), V, jnp.int32).at[:e].set(cols).reshape(NW, EPW)
    verts = vertices[0].astype(jnp.float32)
    verts_t = jnp.zeros((3, VPAD), jnp.float32).at[:, :V].set(verts.T)
    verts_t3 = jnp.zeros((3, ACC_ROWS), jnp.float32).at[:, :V].set(verts.T)
    acc_zeros = jnp.zeros((ACC_F,), jnp.float32)

    mesh = plsc.VectorSubcoreMesh(core_axis_name="c", subcore_axis_name="s")
    sc_params = pltpu.CompilerParams(
        use_tc_tiling_on_sc=False, needs_layout_passes=False)

    k1 = pl.kernel(
        _k1_body,
        out_type=jax.ShapeDtypeStruct((T_SIZE,), jnp.int32),
        mesh=mesh,
        compiler_params=sc_params,
        scratch_types=[
            pltpu.VMEM((EPW,), jnp.int32),
            pltpu.VMEM((EPW,), jnp.int32),
            pltpu.VMEM((EPW,), jnp.int32),
            pltpu.VMEM((EPW,), jnp.int32),
            pltpu.SemaphoreType.DMA,
        ],
    )
    table = k1(rows2, cols2)

    k2 = pl.kernel(
        _k2_body,
        out_type=jax.ShapeDtypeStruct((NW, ACC_F), jnp.float32),
        mesh=mesh,
        compiler_params=sc_params,
        scratch_types=[
            pltpu.VMEM((EPW,), jnp.int32),
            pltpu.VMEM((EPW,), jnp.int32),
            pltpu.VMEM((EPW,), jnp.int32),
            pltpu.VMEM((EPW,), jnp.int32),
            pltpu.VMEM((3, VPAD), jnp.float32),
            pltpu.VMEM((ACC_F,), jnp.float32),
            pltpu.SemaphoreType.DMA,
        ],
    )
    partials = k2(rows2, cols2, table, verts_t, acc_zeros)

    out = pl.pallas_call(
        _k3_body,
        out_shape=jax.ShapeDtypeStruct((1, 1), jnp.float32),
    )(partials, verts_t3)
    return out[0, 0]


# speedup vs baseline: 1.8108x; 1.0151x over previous
import jax
import jax.numpy as jnp
from jax import lax
from jax.experimental import pallas as pl
from jax.experimental.pallas import tpu as pltpu
from jax.experimental.pallas import tpu_sc as plsc

V = 10000
WEIGHT = 0.1
NC, NS, L = 2, 16, 16
NW = NC * NS
EPW = 1920
E_PAD = NW * EPW
T_SIZE = V * V + V + 8
DUMMY = V
ACC_ROWS = 10112
ACC_F = 4 * ACC_ROWS
VPAD = 10016


def _fill_keys(rows_v, cols_v, keys_v):
    @pl.loop(0, EPW // L)
    def _(i):
        sl = pl.ds(i * L, L)
        r = rows_v[sl]
        c = cols_v[sl]
        keys_v[sl] = jnp.minimum(r, c) * V + jnp.maximum(r, c)


def _k1_body(rows_hbm, cols_hbm, t_hbm, rows_v, cols_v, keys_v, eid_v,
             sem, sem2):
    wid = lax.axis_index("s") * NC + lax.axis_index("c")
    in1 = pltpu.async_copy(rows_hbm.at[wid], rows_v, sem)
    in2 = pltpu.async_copy(cols_hbm.at[wid], cols_v, sem2)
    base = wid * EPW
    iota = lax.iota(jnp.int32, L)

    @pl.loop(0, EPW // L)
    def _(i):
        eid_v[pl.ds(i * L, L)] = base + i * L + iota

    in1.wait()
    in2.wait()
    _fill_keys(rows_v, cols_v, keys_v)
    pltpu.async_copy(eid_v, t_hbm.at[keys_v], sem).wait()


def _k2_body(rows_hbm, cols_hbm, t_hbm, verts_hbm, zeros_hbm, out_hbm,
             rows_v, cols_v, keys_v, w_v, verts_v, acc_v, sem):
    wid = lax.axis_index("s") * NC + lax.axis_index("c")
    pltpu.sync_copy(rows_hbm.at[wid], rows_v)
    pltpu.sync_copy(cols_hbm.at[wid], cols_v)

    _fill_keys(rows_v, cols_v, keys_v)
    gather = pltpu.async_copy(t_hbm.at[keys_v], w_v, sem)
    pltpu.sync_copy(verts_hbm, verts_v)
    pltpu.sync_copy(zeros_hbm, acc_v)
    gather.wait()

    base = wid * EPW
    iota = lax.iota(jnp.int32, L)
    ones = jnp.ones((L,), jnp.float32)

    @pl.loop(0, EPW // L)
    def _(i):
        sl = pl.ds(i * L, L)
        r = rows_v[sl]
        c = cols_v[sl]
        w = w_v[sl]
        eid = base + i * L + iota
        canonical = w == eid
        rr = jnp.where(canonical, r, DUMMY)
        cc = jnp.where(canonical & (r != c), c, DUMMY)
        for k in range(3):
            kf = jnp.full((L,), k, jnp.int32)
            vc = plsc.load_gather(verts_v, [kf, c])
            plsc.addupdate_scatter(acc_v, [rr + k * ACC_ROWS], vc)
            vr = plsc.load_gather(verts_v, [kf, r])
            plsc.addupdate_scatter(acc_v, [cc + k * ACC_ROWS], vr)
        plsc.addupdate_scatter(acc_v, [rr + 3 * ACC_ROWS], ones)
        plsc.addupdate_scatter(acc_v, [cc + 3 * ACC_ROWS], ones)

    pltpu.sync_copy(acc_v, out_hbm.at[wid])


def _k3_body(partials_ref, verts_ref, out_ref):
    a = jnp.sum(partials_ref[...], axis=0, keepdims=True)
    deg = a[:, 3 * ACC_ROWS:4 * ACC_ROWS]
    valid = lax.broadcasted_iota(jnp.int32, (1, ACC_ROWS), 1) < V
    total = jnp.zeros((), jnp.float32)
    for k in range(3):
        s = a[:, k * ACC_ROWS:(k + 1) * ACC_ROWS]
        vk = verts_ref[k:k + 1, :]
        r = jnp.where(valid, deg * vk - s, 0.0)
        total = total + jnp.sum(r * r)
    out_ref[...] = jnp.full((1, 1), (WEIGHT / V) * total, jnp.float32)


@jax.jit
def kernel(vertices, faces):
    src_sel = jnp.array([0, 0, 1])
    dst_sel = jnp.array([1, 2, 2])
    rows = faces[:, src_sel].reshape(-1).astype(jnp.int32)
    cols = faces[:, dst_sel].reshape(-1).astype(jnp.int32)
    e = rows.shape[0]
    rows2 = jnp.full((E_PAD,), V, jnp.int32).at[:e].set(rows).reshape(NW, EPW)
    cols2 = jnp.full((E_PAD,), V, jnp.int32).at[:e].set(cols).reshape(NW, EPW)
    verts = vertices[0].astype(jnp.float32)
    verts_t = jnp.zeros((3, VPAD), jnp.float32).at[:, :V].set(verts.T)
    verts_t3 = jnp.zeros((3, ACC_ROWS), jnp.float32).at[:, :V].set(verts.T)
    acc_zeros = jnp.zeros((ACC_F,), jnp.float32)

    mesh = plsc.VectorSubcoreMesh(core_axis_name="c", subcore_axis_name="s")
    sc_params = pltpu.CompilerParams(
        use_tc_tiling_on_sc=False, needs_layout_passes=False)

    k1 = pl.kernel(
        _k1_body,
        out_type=jax.ShapeDtypeStruct((T_SIZE,), jnp.int32),
        mesh=mesh,
        compiler_params=sc_params,
        scratch_types=[
            pltpu.VMEM((EPW,), jnp.int32),
            pltpu.VMEM((EPW,), jnp.int32),
            pltpu.VMEM((EPW,), jnp.int32),
            pltpu.VMEM((EPW,), jnp.int32),
            pltpu.SemaphoreType.DMA,
            pltpu.SemaphoreType.DMA,
        ],
    )
    table = k1(rows2, cols2)

    k2 = pl.kernel(
        _k2_body,
        out_type=jax.ShapeDtypeStruct((NW, ACC_F), jnp.float32),
        mesh=mesh,
        compiler_params=sc_params,
        scratch_types=[
            pltpu.VMEM((EPW,), jnp.int32),
            pltpu.VMEM((EPW,), jnp.int32),
            pltpu.VMEM((EPW,), jnp.int32),
            pltpu.VMEM((EPW,), jnp.int32),
            pltpu.VMEM((3, VPAD), jnp.float32),
            pltpu.VMEM((ACC_F,), jnp.float32),
            pltpu.SemaphoreType.DMA,
        ],
    )
    partials = k2(rows2, cols2, table, verts_t, acc_zeros)

    out = pl.pallas_call(
        _k3_body,
        out_shape=jax.ShapeDtypeStruct((1, 1), jnp.float32),
    )(partials, verts_t3)
    return out[0, 0]
